# baseline (speedup 1.0000x reference)
import functools

import jax
import jax.numpy as jnp
from jax import lax
from jax.experimental import pallas as pl
from jax.experimental.pallas import tpu as pltpu
from jax.experimental.pallas import tpu_sc as plsc

_N = 100000
_E = 3200000
_F = 128
_H = 16
_C = 40
_G = 128

_NC = 2
_NS = 16
_NW = _NC * _NS

_L = 128
_K = 8
_EROWS = _E // _L
_EROWS_PAD = 25088
_WROWS = _EROWS_PAD // _NW
_NCH = _WROWS // _K

_NP = 100096
_RPT = _NP // _NS


def _seg_kernel(with_deg):
  mesh = plsc.VectorSubcoreMesh(core_axis_name="c", subcore_axis_name="s")
  out_type = [jax.ShapeDtypeStruct((_NC, _NP, _H), jnp.float32)]
  if with_deg:
    out_type.append(jax.ShapeDtypeStruct((_NC * _NP,), jnp.float32))
  scratch = [
      pltpu.VMEM((_K, _L), jnp.int32),
      pltpu.VMEM((_K, _L), jnp.int32),
      pltpu.VMEM((_K, _L, _H), jnp.float32),
      pltpu.VMEM_SHARED((_NP, _H), jnp.float32),
      pltpu.SemaphoreType.DMA,
      pltpu.SemaphoreType.DMA,
      pltpu.SemaphoreType.DMA,
      pltpu.SemaphoreType.DMA,
  ]
  if with_deg:
    scratch += [
        pltpu.VMEM((_L,), jnp.float32),
        pltpu.VMEM_SHARED((_NP,), jnp.float32),
        pltpu.SemaphoreType.DMA,
    ]
  _KH = _K // 2

  def body(h_hbm, src_hbm, dst_hbm, z2d_hbm, *rest):
    if with_deg:
      (z1d_hbm, acc_out, deg_out, srcv, dstv, rows, acc_sh,
       gsem, isem, ssemA, ssemB, onesv, deg_sh, dsem) = rest
    else:
      acc_out, srcv, dstv, rows, acc_sh, gsem, isem, ssemA, ssemB = rest
    cid = lax.axis_index("c")
    sid = lax.axis_index("s")
    wid = sid * _NC + cid

    tile_r0 = sid * _RPT
    pltpu.sync_copy(z2d_hbm.at[pl.ds(tile_r0, _RPT)],
                    acc_sh.at[pl.ds(tile_r0, _RPT)])
    if with_deg:
      pltpu.sync_copy(z1d_hbm.at[pl.ds(tile_r0, _RPT)],
                      deg_sh.at[pl.ds(tile_r0, _RPT)])
      for i in range(_L // _H):
        onesv[pl.ds(i * _H, _H)] = jnp.ones((_H,), jnp.float32)
    plsc.subcore_barrier()

    row0 = wid * _WROWS

    def drain(n0, n1, sem):
      for j in range(n0, n1):
        pltpu.make_async_copy(z2d_hbm.at[pl.ds(0, _L)], rows.at[j], sem).wait()

    def chunk(c, carry):
      @pl.when(c > 0)
      def _():
        drain(0, _KH, ssemA)
        drain(_KH, _K, ssemB)
        if with_deg:
          for j in range(_K):
            pltpu.make_async_copy(z1d_hbm.at[pl.ds(0, _L)], onesv, dsem).wait()
      base = row0 + c * _K
      di = pltpu.async_copy(src_hbm.at[pl.ds(base, _K)], srcv, isem)
      dj = pltpu.async_copy(dst_hbm.at[pl.ds(base, _K)], dstv, isem)
      di.wait()
      dj.wait()
      if with_deg:
        for j in range(_K):
          pltpu.async_copy(onesv, deg_sh.at[dstv.at[j]], dsem, add=True)
      gA = [pltpu.async_copy(h_hbm.at[srcv.at[j]], rows.at[j], gsem)
            for j in range(_KH)]
      for d in gA:
        d.wait()
      for j in range(_KH):
        pltpu.async_copy(rows.at[j], acc_sh.at[dstv.at[j]], ssemA, add=True)
      gB = [pltpu.async_copy(h_hbm.at[srcv.at[j]], rows.at[j], gsem)
            for j in range(_KH, _K)]
      for d in gB:
        d.wait()
      for j in range(_KH, _K):
        pltpu.async_copy(rows.at[j], acc_sh.at[dstv.at[j]], ssemB, add=True)
      return carry
    lax.fori_loop(0, _NCH, chunk, 0)

    drain(0, _KH, ssemA)
    drain(_KH, _K, ssemB)
    if with_deg:
      for j in range(_K):
        pltpu.make_async_copy(z1d_hbm.at[pl.ds(0, _L)], onesv, dsem).wait()

    plsc.subcore_barrier()

    pltpu.sync_copy(acc_sh.at[pl.ds(tile_r0, _RPT)],
                    acc_out.at[cid, pl.ds(tile_r0, _RPT)])
    if with_deg:
      pltpu.sync_copy(deg_sh.at[pl.ds(tile_r0, _RPT)],
                      deg_out.at[pl.ds(cid * _NP + tile_r0, _RPT)])

  return pl.kernel(
      body, out_type=out_type, mesh=mesh, scratch_types=scratch,
      compiler_params=pltpu.CompilerParams(use_tc_tiling_on_sc=False))


_BT = 2000
_NBLK = _N // _BT


def _embed_body(x_ref, emb_ref, out_ref):
  x = x_ref[...]
  m = jnp.max(x, axis=1, keepdims=True)
  iota = lax.broadcasted_iota(jnp.int32, (_BT, _F), 1)
  idx = jnp.min(jnp.where(x >= m, iota, _F), axis=1, keepdims=True)
  onehot = (iota == idx).astype(jnp.float32)
  out_ref[...] = jnp.dot(onehot, emb_ref[...],
                         preferred_element_type=jnp.float32)


def _embed(x, emb):
  return pl.pallas_call(
      _embed_body,
      grid=(_NBLK,),
      in_specs=[
          pl.BlockSpec((_BT, _F), lambda i: (i, 0)),
          pl.BlockSpec((_F, _H), lambda i: (0, 0)),
      ],
      out_specs=pl.BlockSpec((_BT, _H), lambda i: (i, 0)),
      out_shape=jax.ShapeDtypeStruct((_N, _H), jnp.float32),
  )(x, emb)


def _layer_body(relu, acc_ref, deg_ref, h_ref, wl_ref, bl_ref, wr_ref,
                out_ref):
  agg = acc_ref[0] + acc_ref[1]
  cnt = deg_ref[0] + deg_ref[1]
  mean = agg / jnp.maximum(cnt, 1.0)
  h = h_ref[...]
  z = (jnp.dot(mean, wl_ref[...], preferred_element_type=jnp.float32)
       + bl_ref[...]
       + jnp.dot(h, wr_ref[...], preferred_element_type=jnp.float32))
  nrm2 = jnp.sum(z * z, axis=1, keepdims=True)
  z = z * lax.rsqrt(jnp.maximum(nrm2, 1e-24))
  if relu:
    z = jnp.maximum(z, 0.0)
  out_ref[...] = z


def _layer(acc, deg3, h, wl, bl2, wr, co, relu):
  return pl.pallas_call(
      functools.partial(_layer_body, relu),
      grid=(_NBLK,),
      in_specs=[
          pl.BlockSpec((_NC, _BT, _H), lambda i: (0, i, 0)),
          pl.BlockSpec((_NC, _BT, 1), lambda i: (0, i, 0)),
          pl.BlockSpec((_BT, _H), lambda i: (i, 0)),
          pl.BlockSpec((_H, co), lambda i: (0, 0)),
          pl.BlockSpec((1, co), lambda i: (0, 0)),
          pl.BlockSpec((_H, co), lambda i: (0, 0)),
      ],
      out_specs=pl.BlockSpec((_BT, co), lambda i: (i, 0)),
      out_shape=jax.ShapeDtypeStruct((_N, co), jnp.float32),
  )(acc, deg3, h, wl, bl2, wr)


def _pool_body(h_ref, batch_ref, out_ref, acc_ref):
  i = pl.program_id(0)

  @pl.when(i == 0)
  def _():
    acc_ref[...] = jnp.zeros((_G, _C + 1), jnp.float32)

  iota = lax.broadcasted_iota(jnp.int32, (_BT, _G), 1)
  onehot = (batch_ref[...] == iota).astype(jnp.float32)
  ext = jnp.concatenate(
      [h_ref[...], jnp.ones((_BT, 1), jnp.float32)], axis=1)
  acc_ref[...] += lax.dot_general(
      onehot, ext, (((0,), (0,)), ((), ())),
      preferred_element_type=jnp.float32)

  @pl.when(i == _NBLK - 1)
  def _():
    a = acc_ref[...]
    pooled = a[:, :_C] / jnp.maximum(a[:, _C:], 1.0)
    m = jnp.max(pooled, axis=1, keepdims=True)
    e = jnp.exp(pooled - m)
    out_ref[...] = e / jnp.sum(e, axis=1, keepdims=True)


def _pool(h3, batch2d):
  return pl.pallas_call(
      _pool_body,
      grid=(_NBLK,),
      in_specs=[
          pl.BlockSpec((_BT, _C), lambda i: (i, 0)),
          pl.BlockSpec((_BT, 1), lambda i: (i, 0)),
      ],
      out_specs=pl.BlockSpec((_G, _C), lambda i: (0, 0)),
      out_shape=jax.ShapeDtypeStruct((_G, _C), jnp.float32),
      scratch_shapes=[pltpu.VMEM((_G, _C + 1), jnp.float32)],
  )(h3, batch2d)


def kernel(x, edge_index, batch, emb, Wl1, bl1, Wr1, Wl2, bl2, Wr2,
           Wl3, bl3, Wr3):
  src2d = edge_index[0].reshape(_EROWS, _L)
  dst2d = edge_index[1].reshape(_EROWS, _L)
  pad_rows = _EROWS_PAD - _EROWS
  src2d = jnp.concatenate(
      [src2d, jnp.zeros((pad_rows, _L), jnp.int32)], axis=0)
  dst2d = jnp.concatenate(
      [dst2d, jnp.full((pad_rows, _L), _N, jnp.int32)], axis=0)

  seg_deg = _seg_kernel(with_deg=True)
  seg = _seg_kernel(with_deg=False)
  z2d = jnp.zeros((_NP, _H), jnp.float32)
  z1d = jnp.zeros((_NP,), jnp.float32)

  h0 = _embed(x, emb)
  acc1, deg = seg_deg(h0, src2d, dst2d, z2d, z1d)
  deg3 = deg.reshape(_NC, _NP, 1)
  h1 = _layer(acc1, deg3, h0, Wl1, bl1.reshape(1, _H), Wr1, _H, relu=True)
  [acc2] = seg(h1, src2d, dst2d, z2d)
  h2 = _layer(acc2, deg3, h1, Wl2, bl2.reshape(1, _H), Wr2, _H, relu=True)
  [acc3] = seg(h2, src2d, dst2d, z2d)
  h3 = _layer(acc3, deg3, h2, Wl3, bl3.reshape(1, _C), Wr3, _C, relu=False)
  return _pool(h3, batch.reshape(_N, 1))


# speedup vs baseline: 1.0853x; 1.0853x over previous
import functools

import jax
import jax.numpy as jnp
from jax import lax
from jax.experimental import pallas as pl
from jax.experimental.pallas import tpu as pltpu
from jax.experimental.pallas import tpu_sc as plsc

_N = 100000
_E = 3200000
_F = 128
_H = 16
_C = 40
_G = 128

_NC = 2
_NS = 16
_NW = _NC * _NS

_L = 128
_K = 8
_EROWS = _E // _L
_EROWS_PAD = 25088
_WROWS = _EROWS_PAD // _NW
_NCH = _WROWS // _K

_NP = 100096
_RPT = _NP // _NS


def _seg_kernel(with_deg):
  mesh = plsc.VectorSubcoreMesh(core_axis_name="c", subcore_axis_name="s")
  out_type = [jax.ShapeDtypeStruct((_NC, _NP, _H), jnp.float32)]
  if with_deg:
    out_type.append(jax.ShapeDtypeStruct((_NC * _NP,), jnp.float32))
  scratch = [
      pltpu.VMEM((2, _K, _L), jnp.int32),
      pltpu.VMEM((2, _K, _L), jnp.int32),
      pltpu.VMEM((_K, _L, _H), jnp.float32),
      pltpu.VMEM_SHARED((_NP, _H), jnp.float32),
      pltpu.SemaphoreType.DMA,
      pltpu.SemaphoreType.DMA,
      pltpu.SemaphoreType.DMA,
      pltpu.SemaphoreType.DMA,
  ]
  if with_deg:
    scratch += [
        pltpu.VMEM((_L,), jnp.float32),
        pltpu.VMEM_SHARED((_NP,), jnp.float32),
        pltpu.SemaphoreType.DMA,
    ]
  _KH = _K // 2

  def body(h_hbm, src_hbm, dst_hbm, z2d_hbm, *rest):
    if with_deg:
      (z1d_hbm, acc_out, deg_out, srcv, dstv, rows, acc_sh,
       gsem, isem, ssemA, ssemB, onesv, deg_sh, dsem) = rest
    else:
      acc_out, srcv, dstv, rows, acc_sh, gsem, isem, ssemA, ssemB = rest
    cid = lax.axis_index("c")
    sid = lax.axis_index("s")
    wid = sid * _NC + cid

    tile_r0 = sid * _RPT
    pltpu.sync_copy(z2d_hbm.at[pl.ds(tile_r0, _RPT)],
                    acc_sh.at[pl.ds(tile_r0, _RPT)])
    if with_deg:
      pltpu.sync_copy(z1d_hbm.at[pl.ds(tile_r0, _RPT)],
                      deg_sh.at[pl.ds(tile_r0, _RPT)])
      for i in range(_L // _H):
        onesv[pl.ds(i * _H, _H)] = jnp.ones((_H,), jnp.float32)
    plsc.subcore_barrier()

    row0 = wid * _WROWS

    def drain(n0, n1, sem):
      for j in range(n0, n1):
        pltpu.make_async_copy(z2d_hbm.at[pl.ds(0, _L)], rows.at[j], sem).wait()

    def idx_load(c, buf):
      base = row0 + c * _K
      pltpu.async_copy(src_hbm.at[pl.ds(base, _K)], srcv.at[buf], isem)
      pltpu.async_copy(dst_hbm.at[pl.ds(base, _K)], dstv.at[buf], isem)

    idx_load(0, 0)

    def chunk(c, carry):
      par = lax.rem(c, 2)
      pltpu.make_async_copy(src_hbm.at[pl.ds(0, _K)], srcv.at[par], isem).wait()
      pltpu.make_async_copy(dst_hbm.at[pl.ds(0, _K)], dstv.at[par], isem).wait()

      @pl.when(c + 1 < _NCH)
      def _():
        idx_load(c + 1, 1 - par)

      @pl.when(c > 0)
      def _():
        drain(0, _KH, ssemA)
        drain(_KH, _K, ssemB)
        if with_deg:
          for j in range(_K):
            pltpu.make_async_copy(z1d_hbm.at[pl.ds(0, _L)], onesv, dsem).wait()
      if with_deg:
        for j in range(_K):
          pltpu.async_copy(onesv, deg_sh.at[dstv.at[par, j]], dsem, add=True)
      gA = [pltpu.async_copy(h_hbm.at[srcv.at[par, j]], rows.at[j], gsem)
            for j in range(_KH)]
      for d in gA:
        d.wait()
      for j in range(_KH):
        pltpu.async_copy(rows.at[j], acc_sh.at[dstv.at[par, j]], ssemA, add=True)
      gB = [pltpu.async_copy(h_hbm.at[srcv.at[par, j]], rows.at[j], gsem)
            for j in range(_KH, _K)]
      for d in gB:
        d.wait()
      for j in range(_KH, _K):
        pltpu.async_copy(rows.at[j], acc_sh.at[dstv.at[par, j]], ssemB, add=True)
      return carry
    lax.fori_loop(0, _NCH, chunk, 0)

    drain(0, _KH, ssemA)
    drain(_KH, _K, ssemB)
    if with_deg:
      for j in range(_K):
        pltpu.make_async_copy(z1d_hbm.at[pl.ds(0, _L)], onesv, dsem).wait()

    plsc.subcore_barrier()

    pltpu.sync_copy(acc_sh.at[pl.ds(tile_r0, _RPT)],
                    acc_out.at[cid, pl.ds(tile_r0, _RPT)])
    if with_deg:
      pltpu.sync_copy(deg_sh.at[pl.ds(tile_r0, _RPT)],
                      deg_out.at[pl.ds(cid * _NP + tile_r0, _RPT)])

  return pl.kernel(
      body, out_type=out_type, mesh=mesh, scratch_types=scratch,
      compiler_params=pltpu.CompilerParams(use_tc_tiling_on_sc=False))


_BT = 2000
_NBLK = _N // _BT


def _embed_body(x_ref, emb_ref, out_ref):
  x = x_ref[...]
  m = jnp.max(x, axis=1, keepdims=True)
  iota = lax.broadcasted_iota(jnp.int32, (_BT, _F), 1)
  idx = jnp.min(jnp.where(x >= m, iota, _F), axis=1, keepdims=True)
  onehot = (iota == idx).astype(jnp.float32)
  out_ref[...] = jnp.dot(onehot, emb_ref[...],
                         preferred_element_type=jnp.float32)


def _embed(x, emb):
  return pl.pallas_call(
      _embed_body,
      grid=(_NBLK,),
      in_specs=[
          pl.BlockSpec((_BT, _F), lambda i: (i, 0)),
          pl.BlockSpec((_F, _H), lambda i: (0, 0)),
      ],
      out_specs=pl.BlockSpec((_BT, _H), lambda i: (i, 0)),
      out_shape=jax.ShapeDtypeStruct((_N, _H), jnp.float32),
  )(x, emb)


def _layer_body(relu, acc_ref, deg_ref, h_ref, wl_ref, bl_ref, wr_ref,
                out_ref):
  agg = acc_ref[0] + acc_ref[1]
  cnt = deg_ref[0] + deg_ref[1]
  mean = agg / jnp.maximum(cnt, 1.0)
  h = h_ref[...]
  z = (jnp.dot(mean, wl_ref[...], preferred_element_type=jnp.float32)
       + bl_ref[...]
       + jnp.dot(h, wr_ref[...], preferred_element_type=jnp.float32))
  nrm2 = jnp.sum(z * z, axis=1, keepdims=True)
  z = z * lax.rsqrt(jnp.maximum(nrm2, 1e-24))
  if relu:
    z = jnp.maximum(z, 0.0)
  out_ref[...] = z


def _layer(acc, deg3, h, wl, bl2, wr, co, relu):
  return pl.pallas_call(
      functools.partial(_layer_body, relu),
      grid=(_NBLK,),
      in_specs=[
          pl.BlockSpec((_NC, _BT, _H), lambda i: (0, i, 0)),
          pl.BlockSpec((_NC, _BT, 1), lambda i: (0, i, 0)),
          pl.BlockSpec((_BT, _H), lambda i: (i, 0)),
          pl.BlockSpec((_H, co), lambda i: (0, 0)),
          pl.BlockSpec((1, co), lambda i: (0, 0)),
          pl.BlockSpec((_H, co), lambda i: (0, 0)),
      ],
      out_specs=pl.BlockSpec((_BT, co), lambda i: (i, 0)),
      out_shape=jax.ShapeDtypeStruct((_N, co), jnp.float32),
  )(acc, deg3, h, wl, bl2, wr)


def _pool_body(h_ref, batch_ref, out_ref, acc_ref):
  i = pl.program_id(0)

  @pl.when(i == 0)
  def _():
    acc_ref[...] = jnp.zeros((_G, _C + 1), jnp.float32)

  iota = lax.broadcasted_iota(jnp.int32, (_BT, _G), 1)
  onehot = (batch_ref[...] == iota).astype(jnp.float32)
  ext = jnp.concatenate(
      [h_ref[...], jnp.ones((_BT, 1), jnp.float32)], axis=1)
  acc_ref[...] += lax.dot_general(
      onehot, ext, (((0,), (0,)), ((), ())),
      preferred_element_type=jnp.float32)

  @pl.when(i == _NBLK - 1)
  def _():
    a = acc_ref[...]
    pooled = a[:, :_C] / jnp.maximum(a[:, _C:], 1.0)
    m = jnp.max(pooled, axis=1, keepdims=True)
    e = jnp.exp(pooled - m)
    out_ref[...] = e / jnp.sum(e, axis=1, keepdims=True)


def _pool(h3, batch2d):
  return pl.pallas_call(
      _pool_body,
      grid=(_NBLK,),
      in_specs=[
          pl.BlockSpec((_BT, _C), lambda i: (i, 0)),
          pl.BlockSpec((_BT, 1), lambda i: (i, 0)),
      ],
      out_specs=pl.BlockSpec((_G, _C), lambda i: (0, 0)),
      out_shape=jax.ShapeDtypeStruct((_G, _C), jnp.float32),
      scratch_shapes=[pltpu.VMEM((_G, _C + 1), jnp.float32)],
  )(h3, batch2d)


def kernel(x, edge_index, batch, emb, Wl1, bl1, Wr1, Wl2, bl2, Wr2,
           Wl3, bl3, Wr3):
  src2d = edge_index[0].reshape(_EROWS, _L)
  dst2d = edge_index[1].reshape(_EROWS, _L)
  pad_rows = _EROWS_PAD - _EROWS
  src2d = jnp.concatenate(
      [src2d, jnp.zeros((pad_rows, _L), jnp.int32)], axis=0)
  dst2d = jnp.concatenate(
      [dst2d, jnp.full((pad_rows, _L), _N, jnp.int32)], axis=0)

  seg_deg = _seg_kernel(with_deg=True)
  seg = _seg_kernel(with_deg=False)
  z2d = jnp.zeros((_NP, _H), jnp.float32)
  z1d = jnp.zeros((_NP,), jnp.float32)

  h0 = _embed(x, emb)
  acc1, deg = seg_deg(h0, src2d, dst2d, z2d, z1d)
  deg3 = deg.reshape(_NC, _NP, 1)
  h1 = _layer(acc1, deg3, h0, Wl1, bl1.reshape(1, _H), Wr1, _H, relu=True)
  [acc2] = seg(h1, src2d, dst2d, z2d)
  h2 = _layer(acc2, deg3, h1, Wl2, bl2.reshape(1, _H), Wr2, _H, relu=True)
  [acc3] = seg(h2, src2d, dst2d, z2d)
  h3 = _layer(acc3, deg3, h2, Wl3, bl3.reshape(1, _C), Wr3, _C, relu=False)
  return _pool(h3, batch.reshape(_N, 1))


# speedup vs baseline: 1.0971x; 1.0109x over previous
import functools

import jax
import jax.numpy as jnp
from jax import lax
from jax.experimental import pallas as pl
from jax.experimental.pallas import tpu as pltpu
from jax.experimental.pallas import tpu_sc as plsc

_N = 100000
_E = 3200000
_F = 128
_H = 16
_C = 40
_G = 128

_NC = 2
_NS = 16
_NW = _NC * _NS

_L = 128
_K = 8
_EROWS = _E // _L
_EROWS_PAD = 25088
_WROWS = _EROWS_PAD // _NW
_NCH = _WROWS // _K

_NP = 100096
_RPT = _NP // _NS


def _seg_kernel(with_deg):
  mesh = plsc.VectorSubcoreMesh(core_axis_name="c", subcore_axis_name="s")
  out_type = [jax.ShapeDtypeStruct((_NC, _NP, _H), jnp.float32)]
  if with_deg:
    out_type.append(jax.ShapeDtypeStruct((_NC * _NP,), jnp.float32))
  _EC = _K * _L
  _EH = _EC // 2
  scratch = [
      pltpu.VMEM((2, 2, _EH), jnp.int32),
      pltpu.VMEM((2, 2, _EH), jnp.int32),
      pltpu.VMEM((2, _EH, _H), jnp.float32),
      pltpu.VMEM_SHARED((_NP, _H), jnp.float32),
      pltpu.SemaphoreType.DMA,
      pltpu.SemaphoreType.DMA,
      pltpu.SemaphoreType.DMA,
      pltpu.SemaphoreType.DMA,
  ]
  if with_deg:
    scratch += [
        pltpu.VMEM((_EH,), jnp.float32),
        pltpu.VMEM_SHARED((_NP,), jnp.float32),
        pltpu.SemaphoreType.DMA,
    ]

  def body(h_hbm, src_hbm, dst_hbm, z2d_hbm, z3d_hbm, *rest):
    if with_deg:
      (z1d_hbm, zs_hbm, acc_out, deg_out, srcv, dstv, rows, acc_sh,
       gsem, isem, ssemA, ssemB, onesv, deg_sh, dsem) = rest
    else:
      acc_out, srcv, dstv, rows, acc_sh, gsem, isem, ssemA, ssemB = rest
    cid = lax.axis_index("c")
    sid = lax.axis_index("s")
    wid = sid * _NC + cid

    tile_r0 = sid * _RPT
    pltpu.sync_copy(z2d_hbm.at[pl.ds(tile_r0, _RPT)],
                    acc_sh.at[pl.ds(tile_r0, _RPT)])
    if with_deg:
      pltpu.sync_copy(z1d_hbm.at[pl.ds(tile_r0, _RPT)],
                      deg_sh.at[pl.ds(tile_r0, _RPT)])
      for j in range(_EH // _H):
        onesv[pl.ds(j * _H, _H)] = jnp.ones((_H,), jnp.float32)
    plsc.subcore_barrier()

    chunk0 = wid * _NCH

    def chunk(c, carry):
      par = lax.rem(c, 2)
      pltpu.make_async_copy(src_hbm.at[0], srcv.at[par], isem).wait()
      pltpu.make_async_copy(dst_hbm.at[0], dstv.at[par], isem).wait()

      @pl.when(c + 1 < _NCH)
      def _():
        pltpu.async_copy(src_hbm.at[chunk0 + c + 1], srcv.at[1 - par], isem)
        pltpu.async_copy(dst_hbm.at[chunk0 + c + 1], dstv.at[1 - par], isem)

      @pl.when(c > 0)
      def _():
        pltpu.make_async_copy(z3d_hbm, rows.at[0], ssemA).wait()
        pltpu.make_async_copy(z3d_hbm, rows.at[1], ssemB).wait()
        if with_deg:
          pltpu.make_async_copy(zs_hbm, onesv, dsem).wait()
          pltpu.make_async_copy(zs_hbm, onesv, dsem).wait()
      if with_deg:
        pltpu.async_copy(onesv, deg_sh.at[dstv.at[par, 0]], dsem, add=True)
        pltpu.async_copy(onesv, deg_sh.at[dstv.at[par, 1]], dsem, add=True)
      gA = pltpu.async_copy(h_hbm.at[srcv.at[par, 0]], rows.at[0], gsem)
      gA.wait()
      pltpu.async_copy(rows.at[0], acc_sh.at[dstv.at[par, 0]], ssemA, add=True)
      gB = pltpu.async_copy(h_hbm.at[srcv.at[par, 1]], rows.at[1], gsem)
      gB.wait()
      pltpu.async_copy(rows.at[1], acc_sh.at[dstv.at[par, 1]], ssemB, add=True)
      return carry

    pltpu.async_copy(src_hbm.at[chunk0], srcv.at[0], isem)
    pltpu.async_copy(dst_hbm.at[chunk0], dstv.at[0], isem)
    lax.fori_loop(0, _NCH, chunk, 0)

    pltpu.make_async_copy(z3d_hbm, rows.at[0], ssemA).wait()
    pltpu.make_async_copy(z3d_hbm, rows.at[1], ssemB).wait()
    if with_deg:
      pltpu.make_async_copy(zs_hbm, onesv, dsem).wait()
      pltpu.make_async_copy(zs_hbm, onesv, dsem).wait()

    plsc.subcore_barrier()

    pltpu.sync_copy(acc_sh.at[pl.ds(tile_r0, _RPT)],
                    acc_out.at[cid, pl.ds(tile_r0, _RPT)])
    if with_deg:
      pltpu.sync_copy(deg_sh.at[pl.ds(tile_r0, _RPT)],
                      deg_out.at[pl.ds(cid * _NP + tile_r0, _RPT)])

  return pl.kernel(
      body, out_type=out_type, mesh=mesh, scratch_types=scratch,
      compiler_params=pltpu.CompilerParams(use_tc_tiling_on_sc=False))


_BT = 2000
_NBLK = _N // _BT


def _embed_body(x_ref, emb_ref, out_ref):
  x = x_ref[...]
  m = jnp.max(x, axis=1, keepdims=True)
  iota = lax.broadcasted_iota(jnp.int32, (_BT, _F), 1)
  idx = jnp.min(jnp.where(x >= m, iota, _F), axis=1, keepdims=True)
  onehot = (iota == idx).astype(jnp.float32)
  out_ref[...] = jnp.dot(onehot, emb_ref[...],
                         preferred_element_type=jnp.float32)


def _embed(x, emb):
  return pl.pallas_call(
      _embed_body,
      grid=(_NBLK,),
      in_specs=[
          pl.BlockSpec((_BT, _F), lambda i: (i, 0)),
          pl.BlockSpec((_F, _H), lambda i: (0, 0)),
      ],
      out_specs=pl.BlockSpec((_BT, _H), lambda i: (i, 0)),
      out_shape=jax.ShapeDtypeStruct((_N, _H), jnp.float32),
  )(x, emb)


def _layer_body(relu, acc_ref, deg_ref, h_ref, wl_ref, bl_ref, wr_ref,
                out_ref):
  agg = acc_ref[0] + acc_ref[1]
  cnt = deg_ref[0] + deg_ref[1]
  mean = agg / jnp.maximum(cnt, 1.0)
  h = h_ref[...]
  z = (jnp.dot(mean, wl_ref[...], preferred_element_type=jnp.float32)
       + bl_ref[...]
       + jnp.dot(h, wr_ref[...], preferred_element_type=jnp.float32))
  nrm2 = jnp.sum(z * z, axis=1, keepdims=True)
  z = z * lax.rsqrt(jnp.maximum(nrm2, 1e-24))
  if relu:
    z = jnp.maximum(z, 0.0)
  out_ref[...] = z


def _layer(acc, deg3, h, wl, bl2, wr, co, relu):
  return pl.pallas_call(
      functools.partial(_layer_body, relu),
      grid=(_NBLK,),
      in_specs=[
          pl.BlockSpec((_NC, _BT, _H), lambda i: (0, i, 0)),
          pl.BlockSpec((_NC, _BT, 1), lambda i: (0, i, 0)),
          pl.BlockSpec((_BT, _H), lambda i: (i, 0)),
          pl.BlockSpec((_H, co), lambda i: (0, 0)),
          pl.BlockSpec((1, co), lambda i: (0, 0)),
          pl.BlockSpec((_H, co), lambda i: (0, 0)),
      ],
      out_specs=pl.BlockSpec((_BT, co), lambda i: (i, 0)),
      out_shape=jax.ShapeDtypeStruct((_N, co), jnp.float32),
  )(acc, deg3, h, wl, bl2, wr)


def _pool_body(h_ref, batch_ref, out_ref, acc_ref):
  i = pl.program_id(0)

  @pl.when(i == 0)
  def _():
    acc_ref[...] = jnp.zeros((_G, _C + 1), jnp.float32)

  iota = lax.broadcasted_iota(jnp.int32, (_BT, _G), 1)
  onehot = (batch_ref[...] == iota).astype(jnp.float32)
  ext = jnp.concatenate(
      [h_ref[...], jnp.ones((_BT, 1), jnp.float32)], axis=1)
  acc_ref[...] += lax.dot_general(
      onehot, ext, (((0,), (0,)), ((), ())),
      preferred_element_type=jnp.float32)

  @pl.when(i == _NBLK - 1)
  def _():
    a = acc_ref[...]
    pooled = a[:, :_C] / jnp.maximum(a[:, _C:], 1.0)
    m = jnp.max(pooled, axis=1, keepdims=True)
    e = jnp.exp(pooled - m)
    out_ref[...] = e / jnp.sum(e, axis=1, keepdims=True)


def _pool(h3, batch2d):
  return pl.pallas_call(
      _pool_body,
      grid=(_NBLK,),
      in_specs=[
          pl.BlockSpec((_BT, _C), lambda i: (i, 0)),
          pl.BlockSpec((_BT, 1), lambda i: (i, 0)),
      ],
      out_specs=pl.BlockSpec((_G, _C), lambda i: (0, 0)),
      out_shape=jax.ShapeDtypeStruct((_G, _C), jnp.float32),
      scratch_shapes=[pltpu.VMEM((_G, _C + 1), jnp.float32)],
  )(h3, batch2d)


def kernel(x, edge_index, batch, emb, Wl1, bl1, Wr1, Wl2, bl2, Wr2,
           Wl3, bl3, Wr3):
  src2d = edge_index[0].reshape(_EROWS, _L)
  dst2d = edge_index[1].reshape(_EROWS, _L)
  pad_rows = _EROWS_PAD - _EROWS
  ec = _K * _L
  src3d = jnp.concatenate(
      [src2d, jnp.zeros((pad_rows, _L), jnp.int32)], axis=0
  ).reshape(_EROWS_PAD * _L // ec, 2, ec // 2)
  dst3d = jnp.concatenate(
      [dst2d, jnp.full((pad_rows, _L), _N, jnp.int32)], axis=0
  ).reshape(_EROWS_PAD * _L // ec, 2, ec // 2)

  seg_deg = _seg_kernel(with_deg=True)
  seg = _seg_kernel(with_deg=False)
  z2d = jnp.zeros((_NP, _H), jnp.float32)
  z1d = jnp.zeros((_NP,), jnp.float32)
  z3d = jnp.zeros((ec // 2, _H), jnp.float32)
  zs = jnp.zeros((ec // 2,), jnp.float32)

  h0 = _embed(x, emb)
  acc1, deg = seg_deg(h0, src3d, dst3d, z2d, z3d, z1d, zs)
  deg3 = deg.reshape(_NC, _NP, 1)
  h1 = _layer(acc1, deg3, h0, Wl1, bl1.reshape(1, _H), Wr1, _H, relu=True)
  [acc2] = seg(h1, src3d, dst3d, z2d, z3d)
  h2 = _layer(acc2, deg3, h1, Wl2, bl2.reshape(1, _H), Wr2, _H, relu=True)
  [acc3] = seg(h2, src3d, dst3d, z2d, z3d)
  h3 = _layer(acc3, deg3, h2, Wl3, bl3.reshape(1, _C), Wr3, _C, relu=False)
  return _pool(h3, batch.reshape(_N, 1))


# speedup vs baseline: 1.3976x; 1.2739x over previous
import functools

import jax
import jax.numpy as jnp
from jax import lax
from jax.experimental import pallas as pl
from jax.experimental.pallas import tpu as pltpu
from jax.experimental.pallas import tpu_sc as plsc

_N = 100000
_E = 3200000
_F = 128
_H = 16
_C = 40
_G = 128

_NC = 2
_NS = 16
_NW = _NC * _NS

_L = 128
_K = 8
_EROWS = _E // _L
_EROWS_PAD = 25088
_WROWS = _EROWS_PAD // _NW
_NCH = _WROWS // _K

_NP = 100096
_RPT = _NP // _NS


def _seg_kernel(with_deg):
  mesh = plsc.VectorSubcoreMesh(core_axis_name="c", subcore_axis_name="s")
  out_type = [jax.ShapeDtypeStruct((_NC, _NP, _H), jnp.float32)]
  if with_deg:
    out_type.append(jax.ShapeDtypeStruct((_NC * _NP,), jnp.float32))
  _EC = _K * _L
  _EH = _EC // 2
  scratch = [
      pltpu.VMEM((2, 2, _EH), jnp.int32),
      pltpu.VMEM((2, 2, _EH), jnp.int32),
      pltpu.VMEM((2, _EH, _H), jnp.float32),
      pltpu.VMEM_SHARED((_NP, _H), jnp.float32),
      pltpu.SemaphoreType.DMA,
      pltpu.SemaphoreType.DMA,
      pltpu.SemaphoreType.DMA,
      pltpu.SemaphoreType.DMA,
  ]
  if with_deg:
    scratch += [
        pltpu.VMEM((_EH,), jnp.float32),
        pltpu.VMEM_SHARED((_NP,), jnp.float32),
        pltpu.SemaphoreType.DMA,
    ]

  def body(h_hbm, src_hbm, dst_hbm, z2d_hbm, z3d_hbm, *rest):
    if with_deg:
      (z1d_hbm, zs_hbm, acc_out, deg_out, srcv, dstv, rows, acc_sh,
       gsem, isem, ssemA, ssemB, onesv, deg_sh, dsem) = rest
    else:
      acc_out, srcv, dstv, rows, acc_sh, gsem, isem, ssemA, ssemB = rest
    cid = lax.axis_index("c")
    sid = lax.axis_index("s")
    wid = sid * _NC + cid

    tile_r0 = sid * _RPT
    pltpu.sync_copy(z2d_hbm.at[pl.ds(tile_r0, _RPT)],
                    acc_sh.at[pl.ds(tile_r0, _RPT)])
    if with_deg:
      pltpu.sync_copy(z1d_hbm.at[pl.ds(tile_r0, _RPT)],
                      deg_sh.at[pl.ds(tile_r0, _RPT)])
      for j in range(_EH // _H):
        onesv[pl.ds(j * _H, _H)] = jnp.ones((_H,), jnp.float32)
    plsc.subcore_barrier()

    chunk0 = wid * _NCH

    def chunk(c, carry):
      par = lax.rem(c, 2)
      pltpu.make_async_copy(src_hbm.at[0], srcv.at[par], isem).wait()
      pltpu.make_async_copy(dst_hbm.at[0], dstv.at[par], isem).wait()

      @pl.when(c + 1 < _NCH)
      def _():
        pltpu.async_copy(src_hbm.at[chunk0 + c + 1], srcv.at[1 - par], isem)
        pltpu.async_copy(dst_hbm.at[chunk0 + c + 1], dstv.at[1 - par], isem)

      @pl.when(c > 0)
      def _():
        pltpu.make_async_copy(z3d_hbm, rows.at[0], ssemA).wait()
        pltpu.make_async_copy(z3d_hbm, rows.at[1], ssemB).wait()
        if with_deg:
          pltpu.make_async_copy(zs_hbm, onesv, dsem).wait()
          pltpu.make_async_copy(zs_hbm, onesv, dsem).wait()
      if with_deg:
        pltpu.async_copy(onesv, deg_sh.at[dstv.at[par, 0]], dsem, add=True)
        pltpu.async_copy(onesv, deg_sh.at[dstv.at[par, 1]], dsem, add=True)
      gA = pltpu.async_copy(h_hbm.at[srcv.at[par, 0]], rows.at[0], gsem)
      gA.wait()
      pltpu.async_copy(rows.at[0], acc_sh.at[dstv.at[par, 0]], ssemA, add=True)
      gB = pltpu.async_copy(h_hbm.at[srcv.at[par, 1]], rows.at[1], gsem)
      gB.wait()
      pltpu.async_copy(rows.at[1], acc_sh.at[dstv.at[par, 1]], ssemB, add=True)
      return carry

    pltpu.async_copy(src_hbm.at[chunk0], srcv.at[0], isem)
    pltpu.async_copy(dst_hbm.at[chunk0], dstv.at[0], isem)
    lax.fori_loop(0, _NCH, chunk, 0)

    pltpu.make_async_copy(z3d_hbm, rows.at[0], ssemA).wait()
    pltpu.make_async_copy(z3d_hbm, rows.at[1], ssemB).wait()
    if with_deg:
      pltpu.make_async_copy(zs_hbm, onesv, dsem).wait()
      pltpu.make_async_copy(zs_hbm, onesv, dsem).wait()

    plsc.subcore_barrier()

    pltpu.sync_copy(acc_sh.at[pl.ds(tile_r0, _RPT)],
                    acc_out.at[cid, pl.ds(tile_r0, _RPT)])
    if with_deg:
      pltpu.sync_copy(deg_sh.at[pl.ds(tile_r0, _RPT)],
                      deg_out.at[pl.ds(cid * _NP + tile_r0, _RPT)])

  return pl.kernel(
      body, out_type=out_type, mesh=mesh, scratch_types=scratch,
      compiler_params=pltpu.CompilerParams(use_tc_tiling_on_sc=False))


_BT = 2048
_NBLK = (_N + _BT - 1) // _BT


_EB = 256


def _embed_body(x_ref, embb_ref, out_ref):
  xr = x_ref[...].reshape(_EB, 8, _F)
  m = jnp.max(xr, axis=2, keepdims=True)
  iota = lax.broadcasted_iota(jnp.int32, (_EB, 8, _F), 2)
  idx = jnp.min(jnp.where(xr >= m, iota, _F), axis=2, keepdims=True)
  onehot = (iota == idx).astype(jnp.float32).reshape(_EB, 8 * _F)
  out_ref[...] = jnp.dot(onehot, embb_ref[...],
                         preferred_element_type=jnp.float32)


def _embed(xb, embb):
  nblk = (_NPR + _EB - 1) // _EB
  return pl.pallas_call(
      _embed_body,
      grid=(nblk,),
      in_specs=[
          pl.BlockSpec((_EB, 8 * _F), lambda i: (i, 0)),
          pl.BlockSpec((8 * _F, _L), lambda i: (0, 0)),
      ],
      out_specs=pl.BlockSpec((_EB, _L), lambda i: (i, 0)),
      out_shape=jax.ShapeDtypeStruct((_NPR, _L), jnp.float32),
  )(xb, embb)


_PK = 8
_PB = _BT // _PK
_NPR = _N // _PK
_NPPR = _NP * _H // _L


def _layer_body(relu, acc_ref, icnt_ref, h_ref, wl_ref, bl_ref, wr_ref,
                ones_ref, out_ref):
  agg = acc_ref[0] + acc_ref[1]
  z = (jnp.dot(agg, wl_ref[...], preferred_element_type=jnp.float32)
       * icnt_ref[...]
       + bl_ref[...]
       + jnp.dot(h_ref[...], wr_ref[...], preferred_element_type=jnp.float32))
  nrm2 = jnp.dot(z * z, ones_ref[...], preferred_element_type=jnp.float32)
  z = z * lax.rsqrt(jnp.maximum(nrm2, 1e-24))
  if relu:
    z = jnp.maximum(z, 0.0)
  out_ref[...] = z


def _layer(acc, icnt, h, wlb, blb, wrb, onesb, co, relu):
  cop = co * _PK
  return pl.pallas_call(
      functools.partial(_layer_body, relu),
      grid=(_NBLK,),
      in_specs=[
          pl.BlockSpec((_NC, _PB, _L), lambda i: (0, i, 0)),
          pl.BlockSpec((_PB, cop), lambda i: (i, 0)),
          pl.BlockSpec((_PB, _L), lambda i: (i, 0)),
          pl.BlockSpec((_L, cop), lambda i: (0, 0)),
          pl.BlockSpec((1, cop), lambda i: (0, 0)),
          pl.BlockSpec((_L, cop), lambda i: (0, 0)),
          pl.BlockSpec((cop, cop), lambda i: (0, 0)),
      ],
      out_specs=pl.BlockSpec((_PB, cop), lambda i: (i, 0)),
      out_shape=jax.ShapeDtypeStruct((_NPR, cop), jnp.float32),
  )(acc, icnt, h, wlb, blb, wrb, onesb)


def _pool_body(h_ref, batch_ref, out_ref, acc_ref):
  i = pl.program_id(0)

  @pl.when(i == 0)
  def _():
    acc_ref[...] = jnp.zeros((_G, _C + 1), jnp.float32)

  iota = lax.broadcasted_iota(jnp.int32, (_BT, _G), 1)
  rowid = i * _BT + lax.broadcasted_iota(jnp.int32, (_BT, _G), 0)
  onehot = ((batch_ref[...] == iota) & (rowid < _N)).astype(jnp.float32)
  ext = jnp.concatenate(
      [h_ref[...], jnp.ones((_BT, 1), jnp.float32)], axis=1)
  acc_ref[...] += lax.dot_general(
      onehot, ext, (((0,), (0,)), ((), ())),
      preferred_element_type=jnp.float32)

  @pl.when(i == _NBLK - 1)
  def _():
    a = acc_ref[...]
    pooled = a[:, :_C] / jnp.maximum(a[:, _C:], 1.0)
    m = jnp.max(pooled, axis=1, keepdims=True)
    e = jnp.exp(pooled - m)
    out_ref[...] = e / jnp.sum(e, axis=1, keepdims=True)


def _pool(h3, batch2d):
  return pl.pallas_call(
      _pool_body,
      grid=(_NBLK,),
      in_specs=[
          pl.BlockSpec((_BT, _C), lambda i: (i, 0)),
          pl.BlockSpec((_BT, 1), lambda i: (i, 0)),
      ],
      out_specs=pl.BlockSpec((_G, _C), lambda i: (0, 0)),
      out_shape=jax.ShapeDtypeStruct((_G, _C), jnp.float32),
      scratch_shapes=[pltpu.VMEM((_G, _C + 1), jnp.float32)],
  )(h3, batch2d)


def kernel(x, edge_index, batch, emb, Wl1, bl1, Wr1, Wl2, bl2, Wr2,
           Wl3, bl3, Wr3):
  src2d = edge_index[0].reshape(_EROWS, _L)
  dst2d = edge_index[1].reshape(_EROWS, _L)
  pad_rows = _EROWS_PAD - _EROWS
  ec = _K * _L
  src3d = jnp.concatenate(
      [src2d, jnp.zeros((pad_rows, _L), jnp.int32)], axis=0
  ).reshape(_EROWS_PAD * _L // ec, 2, ec // 2)
  dst3d = jnp.concatenate(
      [dst2d, jnp.full((pad_rows, _L), _N, jnp.int32)], axis=0
  ).reshape(_EROWS_PAD * _L // ec, 2, ec // 2)

  seg_deg = _seg_kernel(with_deg=True)
  seg = _seg_kernel(with_deg=False)
  z2d = jnp.zeros((_NP, _H), jnp.float32)
  z1d = jnp.zeros((_NP,), jnp.float32)
  z3d = jnp.zeros((ec // 2, _H), jnp.float32)
  zs = jnp.zeros((ec // 2,), jnp.float32)

  eye8 = jnp.eye(_PK, dtype=jnp.float32)
  wl1b = jnp.kron(eye8, Wl1)
  wr1b = jnp.kron(eye8, Wr1)
  wl2b = jnp.kron(eye8, Wl2)
  wr2b = jnp.kron(eye8, Wr2)
  wl3b = jnp.kron(eye8, Wl3)
  wr3b = jnp.kron(eye8, Wr3)
  bl1b = jnp.tile(bl1, _PK).reshape(1, _L)
  bl2b = jnp.tile(bl2, _PK).reshape(1, _L)
  bl3b = jnp.tile(bl3, _PK).reshape(1, _C * _PK)
  ones16b = jnp.kron(eye8, jnp.ones((_H, _H), jnp.float32))
  ones40b = jnp.kron(eye8, jnp.ones((_C, _C), jnp.float32))

  h0p = _embed(x.reshape(_NPR, 8 * _F), jnp.kron(eye8, emb))
  acc1, deg = seg_deg(h0p.reshape(_N, _H), src3d, dst3d, z2d, z3d, z1d, zs)
  d = deg[:_NP] + deg[_NP:]
  icn = 1.0 / jnp.maximum(d[:_N], 1.0)
  icnt16 = jnp.repeat(icn, _H).reshape(_NPR, _L)
  icnt40 = jnp.repeat(icn, _C).reshape(_NPR, _C * _PK)
  h1p = _layer(acc1.reshape(_NC, _NPPR, _L), icnt16, h0p, wl1b, bl1b, wr1b,
               ones16b, _H, relu=True)
  [acc2] = seg(h1p.reshape(_N, _H), src3d, dst3d, z2d, z3d)
  h2p = _layer(acc2.reshape(_NC, _NPPR, _L), icnt16, h1p, wl2b, bl2b, wr2b,
               ones16b, _H, relu=True)
  [acc3] = seg(h2p.reshape(_N, _H), src3d, dst3d, z2d, z3d)
  h3p = _layer(acc3.reshape(_NC, _NPPR, _L), icnt40, h2p, wl3b, bl3b, wr3b,
               ones40b, _C, relu=False)
  return _pool(h3p.reshape(_N, _C), batch.reshape(_N, 1))


# speedup vs baseline: 1.4033x; 1.0041x over previous
import functools

import jax
import jax.numpy as jnp
from jax import lax
from jax.experimental import pallas as pl
from jax.experimental.pallas import tpu as pltpu
from jax.experimental.pallas import tpu_sc as plsc

_N = 100000
_E = 3200000
_F = 128
_H = 16
_C = 40
_G = 128

_NC = 2
_NS = 16
_NW = _NC * _NS

_L = 128
_K = 8
_EROWS = _E // _L
_EROWS_PAD = 25088
_WROWS = _EROWS_PAD // _NW
_NCH = _WROWS // _K

_NP = 100096
_RPT = _NP // _NS


def _seg_kernel(with_deg):
  mesh = plsc.VectorSubcoreMesh(core_axis_name="c", subcore_axis_name="s")
  out_type = [jax.ShapeDtypeStruct((_NC, _NP, _H), jnp.float32)]
  if with_deg:
    out_type.append(jax.ShapeDtypeStruct((_NC * _NP,), jnp.float32))
  _EC = _K * _L
  _EH = _EC // 2
  scratch = [
      pltpu.VMEM((2, 2, _EH), jnp.int32),
      pltpu.VMEM((2, 2, _EH), jnp.int32),
      pltpu.VMEM((2, _EH, _H), jnp.float32),
      pltpu.VMEM_SHARED((_NP, _H), jnp.float32),
      pltpu.SemaphoreType.DMA,
      pltpu.SemaphoreType.DMA,
      pltpu.SemaphoreType.DMA,
      pltpu.SemaphoreType.DMA,
  ]
  if with_deg:
    scratch += [
        pltpu.VMEM((_EH,), jnp.float32),
        pltpu.VMEM_SHARED((_NP,), jnp.float32),
        pltpu.SemaphoreType.DMA,
    ]

  def body(h_hbm, src_hbm, dst_hbm, z2d_hbm, z3d_hbm, *rest):
    if with_deg:
      (z1d_hbm, zs_hbm, acc_out, deg_out, srcv, dstv, rows, acc_sh,
       gsem, isem, ssemA, ssemB, onesv, deg_sh, dsem) = rest
    else:
      acc_out, srcv, dstv, rows, acc_sh, gsem, isem, ssemA, ssemB = rest
    cid = lax.axis_index("c")
    sid = lax.axis_index("s")
    wid = sid * _NC + cid

    tile_r0 = sid * _RPT
    pltpu.sync_copy(z2d_hbm.at[pl.ds(tile_r0, _RPT)],
                    acc_sh.at[pl.ds(tile_r0, _RPT)])
    if with_deg:
      pltpu.sync_copy(z1d_hbm.at[pl.ds(tile_r0, _RPT)],
                      deg_sh.at[pl.ds(tile_r0, _RPT)])
      for j in range(_EH // _H):
        onesv[pl.ds(j * _H, _H)] = jnp.ones((_H,), jnp.float32)
    plsc.subcore_barrier()

    chunk0 = wid * _NCH

    def chunk(c, carry):
      par = lax.rem(c, 2)
      pltpu.make_async_copy(src_hbm.at[0], srcv.at[par], isem).wait()
      pltpu.make_async_copy(dst_hbm.at[0], dstv.at[par], isem).wait()

      @pl.when(c + 1 < _NCH)
      def _():
        pltpu.async_copy(src_hbm.at[chunk0 + c + 1], srcv.at[1 - par], isem)
        pltpu.async_copy(dst_hbm.at[chunk0 + c + 1], dstv.at[1 - par], isem)

      @pl.when(c > 0)
      def _():
        pltpu.make_async_copy(z3d_hbm, rows.at[0], ssemA).wait()
        pltpu.make_async_copy(z3d_hbm, rows.at[1], ssemB).wait()
        if with_deg:
          pltpu.make_async_copy(zs_hbm, onesv, dsem).wait()
          pltpu.make_async_copy(zs_hbm, onesv, dsem).wait()
      if with_deg:
        pltpu.async_copy(onesv, deg_sh.at[dstv.at[par, 0]], dsem, add=True)
        pltpu.async_copy(onesv, deg_sh.at[dstv.at[par, 1]], dsem, add=True)
      gA = pltpu.async_copy(h_hbm.at[srcv.at[par, 0]], rows.at[0], gsem)
      gA.wait()
      pltpu.async_copy(rows.at[0], acc_sh.at[dstv.at[par, 0]], ssemA, add=True)
      gB = pltpu.async_copy(h_hbm.at[srcv.at[par, 1]], rows.at[1], gsem)
      gB.wait()
      pltpu.async_copy(rows.at[1], acc_sh.at[dstv.at[par, 1]], ssemB, add=True)
      return carry

    pltpu.async_copy(src_hbm.at[chunk0], srcv.at[0], isem)
    pltpu.async_copy(dst_hbm.at[chunk0], dstv.at[0], isem)
    lax.fori_loop(0, _NCH, chunk, 0)

    pltpu.make_async_copy(z3d_hbm, rows.at[0], ssemA).wait()
    pltpu.make_async_copy(z3d_hbm, rows.at[1], ssemB).wait()
    if with_deg:
      pltpu.make_async_copy(zs_hbm, onesv, dsem).wait()
      pltpu.make_async_copy(zs_hbm, onesv, dsem).wait()

    plsc.subcore_barrier()

    pltpu.sync_copy(acc_sh.at[pl.ds(tile_r0, _RPT)],
                    acc_out.at[cid, pl.ds(tile_r0, _RPT)])
    if with_deg:
      pltpu.sync_copy(deg_sh.at[pl.ds(tile_r0, _RPT)],
                      deg_out.at[pl.ds(cid * _NP + tile_r0, _RPT)])

  return pl.kernel(
      body, out_type=out_type, mesh=mesh, scratch_types=scratch,
      compiler_params=pltpu.CompilerParams(use_tc_tiling_on_sc=False))


_BT = 2048
_NBLK = (_N + _BT - 1) // _BT


_EB = 256


def _embed_body(x_ref, embb_ref, out_ref):
  xr = x_ref[...].reshape(_EB, 8, _F)
  m = jnp.max(xr, axis=2, keepdims=True)
  iota = lax.broadcasted_iota(jnp.int32, (_EB, 8, _F), 2)
  idx = jnp.min(jnp.where(xr >= m, iota, _F), axis=2, keepdims=True)
  onehot = (iota == idx).astype(jnp.float32).reshape(_EB, 8 * _F)
  out_ref[...] = jnp.dot(onehot, embb_ref[...],
                         preferred_element_type=jnp.float32)


def _embed(xb, embb):
  nblk = (_NPR + _EB - 1) // _EB
  return pl.pallas_call(
      _embed_body,
      grid=(nblk,),
      in_specs=[
          pl.BlockSpec((_EB, 8 * _F), lambda i: (i, 0)),
          pl.BlockSpec((8 * _F, _L), lambda i: (0, 0)),
      ],
      out_specs=pl.BlockSpec((_EB, _L), lambda i: (i, 0)),
      out_shape=jax.ShapeDtypeStruct((_NPR, _L), jnp.float32),
  )(xb, embb)


_PK = 8
_PB = _BT // _PK
_NPR = _N // _PK
_NPPR = _NP * _H // _L


def _layer_body(relu, acc_ref, icnt_ref, h_ref, wl_ref, bl_ref, wr_ref,
                ones_ref, out_ref):
  agg = acc_ref[0] + acc_ref[1]
  z = (jnp.dot(agg, wl_ref[...], preferred_element_type=jnp.float32)
       * icnt_ref[...]
       + bl_ref[...]
       + jnp.dot(h_ref[...], wr_ref[...], preferred_element_type=jnp.float32))
  nrm2 = jnp.dot(z * z, ones_ref[...], preferred_element_type=jnp.float32)
  z = z * lax.rsqrt(jnp.maximum(nrm2, 1e-24))
  if relu:
    z = jnp.maximum(z, 0.0)
  out_ref[...] = z


def _layer(acc, icnt, h, wlb, blb, wrb, onesb, co, relu):
  cop = co * _PK
  return pl.pallas_call(
      functools.partial(_layer_body, relu),
      grid=(_NBLK,),
      in_specs=[
          pl.BlockSpec((_NC, _PB, _L), lambda i: (0, i, 0)),
          pl.BlockSpec((_PB, cop), lambda i: (i, 0)),
          pl.BlockSpec((_PB, _L), lambda i: (i, 0)),
          pl.BlockSpec((_L, cop), lambda i: (0, 0)),
          pl.BlockSpec((1, cop), lambda i: (0, 0)),
          pl.BlockSpec((_L, cop), lambda i: (0, 0)),
          pl.BlockSpec((cop, cop), lambda i: (0, 0)),
      ],
      out_specs=pl.BlockSpec((_PB, cop), lambda i: (i, 0)),
      out_shape=jax.ShapeDtypeStruct((_NPR, cop), jnp.float32),
  )(acc, icnt, h, wlb, blb, wrb, onesb)


def _pool_body(nblk, h_ref, batch_ref, out_ref, acc_ref):
  i = pl.program_id(0)

  @pl.when(i == 0)
  def _():
    acc_ref[...] = jnp.zeros((_G, _C + 1), jnp.float32)

  hb = h_ref[...].reshape(_EB, 8, _C)
  iota = lax.broadcasted_iota(jnp.int32, (_EB, 8, _G), 2)
  rowid = (i * _EB + lax.broadcasted_iota(jnp.int32, (_EB, 8, _G), 0)) * 8 \
      + lax.broadcasted_iota(jnp.int32, (_EB, 8, _G), 1)
  onehot = ((batch_ref[...][:, :, None] == iota)
            & (rowid < _N)).astype(jnp.float32)
  ext = jnp.concatenate(
      [hb, jnp.ones((_EB, 8, 1), jnp.float32)], axis=2)
  part = lax.dot_general(
      onehot[:, 0, :], ext[:, 0, :], (((0,), (0,)), ((), ())),
      preferred_element_type=jnp.float32)
  for a in range(1, 8):
    part += lax.dot_general(
        onehot[:, a, :], ext[:, a, :], (((0,), (0,)), ((), ())),
        preferred_element_type=jnp.float32)
  acc_ref[...] += part

  @pl.when(i == nblk - 1)
  def _():
    a = acc_ref[...]
    pooled = a[:, :_C] / jnp.maximum(a[:, _C:], 1.0)
    m = jnp.max(pooled, axis=1, keepdims=True)
    e = jnp.exp(pooled - m)
    out_ref[...] = e / jnp.sum(e, axis=1, keepdims=True)


def _pool(h3p, batchp):
  nblk = (_NPR + _EB - 1) // _EB
  return pl.pallas_call(
      functools.partial(_pool_body, nblk),
      grid=(nblk,),
      in_specs=[
          pl.BlockSpec((_EB, _C * 8), lambda i: (i, 0)),
          pl.BlockSpec((_EB, 8), lambda i: (i, 0)),
      ],
      out_specs=pl.BlockSpec((_G, _C), lambda i: (0, 0)),
      out_shape=jax.ShapeDtypeStruct((_G, _C), jnp.float32),
      scratch_shapes=[pltpu.VMEM((_G, _C + 1), jnp.float32)],
  )(h3p, batchp)


def kernel(x, edge_index, batch, emb, Wl1, bl1, Wr1, Wl2, bl2, Wr2,
           Wl3, bl3, Wr3):
  src2d = edge_index[0].reshape(_EROWS, _L)
  dst2d = edge_index[1].reshape(_EROWS, _L)
  pad_rows = _EROWS_PAD - _EROWS
  ec = _K * _L
  src3d = jnp.concatenate(
      [src2d, jnp.zeros((pad_rows, _L), jnp.int32)], axis=0
  ).reshape(_EROWS_PAD * _L // ec, 2, ec // 2)
  dst3d = jnp.concatenate(
      [dst2d, jnp.full((pad_rows, _L), _N, jnp.int32)], axis=0
  ).reshape(_EROWS_PAD * _L // ec, 2, ec // 2)

  seg_deg = _seg_kernel(with_deg=True)
  seg = _seg_kernel(with_deg=False)
  z2d = jnp.zeros((_NP, _H), jnp.float32)
  z1d = jnp.zeros((_NP,), jnp.float32)
  z3d = jnp.zeros((ec // 2, _H), jnp.float32)
  zs = jnp.zeros((ec // 2,), jnp.float32)

  eye8 = jnp.eye(_PK, dtype=jnp.float32)
  wl1b = jnp.kron(eye8, Wl1)
  wr1b = jnp.kron(eye8, Wr1)
  wl2b = jnp.kron(eye8, Wl2)
  wr2b = jnp.kron(eye8, Wr2)
  wl3b = jnp.kron(eye8, Wl3)
  wr3b = jnp.kron(eye8, Wr3)
  bl1b = jnp.tile(bl1, _PK).reshape(1, _L)
  bl2b = jnp.tile(bl2, _PK).reshape(1, _L)
  bl3b = jnp.tile(bl3, _PK).reshape(1, _C * _PK)
  ones16b = jnp.kron(eye8, jnp.ones((_H, _H), jnp.float32))
  ones40b = jnp.kron(eye8, jnp.ones((_C, _C), jnp.float32))

  h0p = _embed(x.reshape(_NPR, 8 * _F), jnp.kron(eye8, emb))
  acc1, deg = seg_deg(h0p.reshape(_N, _H), src3d, dst3d, z2d, z3d, z1d, zs)
  d = deg[:_NP] + deg[_NP:]
  icn = 1.0 / jnp.maximum(d[:_N], 1.0)
  icnt16 = jnp.repeat(icn, _H).reshape(_NPR, _L)
  icnt40 = jnp.repeat(icn, _C).reshape(_NPR, _C * _PK)
  h1p = _layer(acc1.reshape(_NC, _NPPR, _L), icnt16, h0p, wl1b, bl1b, wr1b,
               ones16b, _H, relu=True)
  [acc2] = seg(h1p.reshape(_N, _H), src3d, dst3d, z2d, z3d)
  h2p = _layer(acc2.reshape(_NC, _NPPR, _L), icnt16, h1p, wl2b, bl2b, wr2b,
               ones16b, _H, relu=True)
  [acc3] = seg(h2p.reshape(_N, _H), src3d, dst3d, z2d, z3d)
  h3p = _layer(acc3.reshape(_NC, _NPPR, _L), icnt40, h2p, wl3b, bl3b, wr3b,
               ones40b, _C, relu=False)
  return _pool(h3p, batch.reshape(_NPR, _PK))
